# baseline (speedup 1.0000x reference)
import functools

import jax
import jax.numpy as jnp
from jax import lax
from jax.experimental import pallas as pl
from jax.experimental.pallas import tpu as pltpu
from jax.experimental.pallas import tpu_sc as plsc

_N = 10000
_D = 128
_K = 32
_B = 8192
_NW = 32
_W = 320
_NP = _NW * _W
_C = 4
_RC = _C * _K
_CH = _W // _C
_NB = 5
_PPW = _B // _NW
_PC = 64
_PCH = _PPW // _PC

_mesh = plsc.VectorSubcoreMesh(core_axis_name="c", subcore_axis_name="s")


def _wid():
    return lax.axis_index("s") * 2 + lax.axis_index("c")


@functools.partial(
    pl.kernel, mesh=_mesh,
    out_type=jax.ShapeDtypeStruct((_NP, _D), jnp.float32),
    scratch_types=[
        pltpu.VMEM((_CH, _RC), jnp.int32),
        pltpu.VMEM((_NB, _RC, _D), jnp.float32),
        pltpu.VMEM((_C, _D), jnp.float32),
    ] + [pltpu.SemaphoreType.DMA] * _NB,
)
def _layer(nbr_hbm, table_hbm, out_hbm, idx_v, rows_nb_v, acc_v, *sems):
    wid = _wid()
    pltpu.sync_copy(nbr_hbm.at[wid], idx_v)
    bufs = tuple((rows_nb_v.at[b], sems[b]) for b in range(_NB))
    for b in range(_NB):
        pltpu.async_copy(table_hbm.at[idx_v.at[b]], bufs[b][0], bufs[b][1])

    def chunk2(cp, carry):
        for b in range(_NB):
            ci = cp * _NB + b
            rows_v, sem = bufs[b]
            pltpu.make_async_copy(
                table_hbm.at[idx_v.at[ci]], rows_v, sem).wait()
            for j in range(_C):
                def kstep(k2, accs):
                    accs = list(accs)
                    for u in range(8):
                        r = j * _K + k2 * 8 + u
                        for g in range(8):
                            accs[g] = accs[g] + rows_v[r, pl.ds(g * 16, 16)]
                    return tuple(accs)

                accs = lax.fori_loop(
                    0, _K // 8, kstep,
                    tuple(jnp.zeros((16,), jnp.float32) for _ in range(8)))
                for g in range(8):
                    acc_v[j, pl.ds(g * 16, 16)] = accs[g] * (1.0 / _K)
            pltpu.sync_copy(acc_v, out_hbm.at[pl.ds(wid * _W + ci * _C, _C)])

            @pl.when(ci + _NB < _CH)
            def _fire():
                pltpu.async_copy(table_hbm.at[idx_v.at[ci + _NB]], rows_v, sem)

        return carry

    lax.fori_loop(0, _CH // _NB, chunk2, 0)


@functools.partial(
    pl.kernel, mesh=_mesh,
    out_type=jax.ShapeDtypeStruct((_B, 16), jnp.float32),
    scratch_types=[
        pltpu.VMEM((_PCH, _PC), jnp.int32),
        pltpu.VMEM((_PCH, _PC), jnp.int32),
        pltpu.VMEM((2, 4, _PC, _D), jnp.float32),
        pltpu.VMEM((_PC, 16), jnp.float32),
        pltpu.SemaphoreType.DMA,
        pltpu.SemaphoreType.DMA,
    ],
)
def _pairs(src_hbm, dst_hbm, h1_hbm, h2_hbm, out_hbm,
           sidx_v, didx_v, rows_v, out_v, sem0, sem1):
    wid = _wid()
    pltpu.sync_copy(src_hbm.at[wid], sidx_v)
    pltpu.sync_copy(dst_hbm.at[wid], didx_v)
    sems = (sem0, sem1)

    def fire(ci, b):
        sem = sems[b]
        pltpu.async_copy(h1_hbm.at[sidx_v.at[ci]], rows_v.at[b, 0], sem)
        pltpu.async_copy(h1_hbm.at[didx_v.at[ci]], rows_v.at[b, 1], sem)
        pltpu.async_copy(h2_hbm.at[sidx_v.at[ci]], rows_v.at[b, 2], sem)
        pltpu.async_copy(h2_hbm.at[didx_v.at[ci]], rows_v.at[b, 3], sem)

    def drain(ci, b):
        sem = sems[b]
        pltpu.make_async_copy(h1_hbm.at[sidx_v.at[ci]], rows_v.at[b, 0], sem).wait()
        pltpu.make_async_copy(h1_hbm.at[didx_v.at[ci]], rows_v.at[b, 1], sem).wait()
        pltpu.make_async_copy(h2_hbm.at[sidx_v.at[ci]], rows_v.at[b, 2], sem).wait()
        pltpu.make_async_copy(h2_hbm.at[didx_v.at[ci]], rows_v.at[b, 3], sem).wait()

    fire(0, 0)
    fire(1, 1)
    for ci in range(_PCH):
        b = ci % 2
        drain(ci, b)

        def pstep(p, carry2):
            acc = jnp.zeros((16,), jnp.float32)
            for g in range(8):
                sl = pl.ds(g * 16, 16)
                v1 = rows_v[b, 0, p, sl] - rows_v[b, 1, p, sl]
                acc = acc + v1 * v1
                v2 = rows_v[b, 2, p, sl] - rows_v[b, 3, p, sl]
                acc = acc + v2 * v2
            out_v[p, :] = acc
            return carry2

        lax.fori_loop(0, _PC, pstep, 0)
        pltpu.sync_copy(out_v, out_hbm.at[pl.ds(wid * _PPW + ci * _PC, _PC)])
        if ci + 2 < _PCH:
            fire(ci + 2, b)


def _bce_body(d16_ref, lbl_ref, out_ref):
    dsum = jnp.sum(d16_ref[...], axis=1, keepdims=True) * (1.0 / (_D * 2))
    p = jnp.exp(-dsum)
    lbl = lbl_ref[...]
    eps = 1e-7
    t = lbl * jnp.log(p + eps) + (1.0 - lbl) * jnp.log(1.0 - p + eps)
    out_ref[...] = (-jnp.mean(t)).reshape(1, 1)


def kernel(pairs, labels, neighbors, embedding_state):
    nbr3 = jnp.pad(neighbors, ((0, _NP - _N), (0, 0))).reshape(_NW, _CH, _RC)
    h1 = _layer(nbr3, embedding_state)
    h2 = _layer(nbr3, h1)
    src = pairs[:, 0].reshape(_NW, _PCH, _PC)
    dst = pairs[:, 1].reshape(_NW, _PCH, _PC)
    d16 = _pairs(src, dst, h1, h2)
    lblf = labels.astype(jnp.float32).reshape(_B, 1)
    loss = pl.pallas_call(
        _bce_body,
        out_shape=jax.ShapeDtypeStruct((1, 1), jnp.float32),
    )(d16, lblf)
    return loss.reshape(())


# speedup vs baseline: 3.1929x; 3.1929x over previous
import functools

import jax
import jax.numpy as jnp
from jax import lax
from jax.experimental import pallas as pl
from jax.experimental.pallas import tpu as pltpu
from jax.experimental.pallas import tpu_sc as plsc

_N = 10000
_D = 128
_K = 32
_B = 8192
_NT = 16
_W = 640
_NP = _NT * _W
_HW = _D // 4
_C = 4
_RC = _C * _K
_CH = _W // _C
_NW = 32
_PPW = _B // _NW
_PC = 64
_PCH = _PPW // _PC

_mesh = plsc.VectorSubcoreMesh(core_axis_name="c", subcore_axis_name="s")

_TOPMASK = -65536


@functools.partial(
    pl.kernel, mesh=_mesh,
    out_type=[jax.ShapeDtypeStruct((_NP, _D // 2), jnp.float32),
              jax.ShapeDtypeStruct((_NP, _D // 2), jnp.float32),
              jax.ShapeDtypeStruct((_NP, _HW), jnp.int32),
              jax.ShapeDtypeStruct((_NP, _HW), jnp.int32)],
    scratch_types=[
        pltpu.VMEM((_CH, _RC), jnp.int32),
        pltpu.VMEM((_RC, _HW), jnp.int32),
        pltpu.VMEM((_C, _D // 2), jnp.float32),
        pltpu.VMEM((_C, _HW), jnp.int32),
        pltpu.VMEM_SHARED((_NP, _HW), jnp.int32),
        pltpu.SemaphoreType.DMA,
    ],
)
def _layer_k(nbr_hbm, tbl_a_hbm, tbl_b_hbm,
             out_a_hbm, out_b_hbm, o16_a_hbm, o16_b_hbm,
             idx_v, rows_v, accf_v, acc16_v, tbl_sh, sem):
    cid = lax.axis_index("c")
    sid = lax.axis_index("s")

    @pl.when(cid == 0)
    def _stage_a():
        pltpu.sync_copy(tbl_a_hbm.at[pl.ds(sid * _W, _W)],
                        tbl_sh.at[pl.ds(sid * _W, _W)])

    @pl.when(cid == 1)
    def _stage_b():
        pltpu.sync_copy(tbl_b_hbm.at[pl.ds(sid * _W, _W)],
                        tbl_sh.at[pl.ds(sid * _W, _W)])

    pltpu.sync_copy(nbr_hbm.at[sid], idx_v)
    plsc.subcore_barrier()

    def rne_top(u):
        odd = lax.bitwise_and(
            lax.shift_right_logical(u, jnp.int32(16)), jnp.int32(1))
        return u + jnp.int32(0x7FFF) + odd

    def chunk(ci, carry):
        pltpu.async_copy(tbl_sh.at[idx_v.at[ci]], rows_v, sem).wait()
        for j in range(_C):
            for g in range(2):
                acc_lo = jnp.zeros((16,), jnp.float32)
                acc_hi = jnp.zeros((16,), jnp.float32)
                for k in range(_K):
                    w = rows_v[j * _K + k, pl.ds(g * 16, 16)]
                    acc_lo = acc_lo + lax.bitcast_convert_type(
                        lax.shift_left(w, jnp.int32(16)), jnp.float32)
                    acc_hi = acc_hi + lax.bitcast_convert_type(
                        lax.bitwise_and(w, jnp.int32(_TOPMASK)), jnp.float32)
                m_lo = acc_lo * (1.0 / _K)
                m_hi = acc_hi * (1.0 / _K)
                accf_v[j, pl.ds(g * 32, 16)] = m_lo
                accf_v[j, pl.ds(g * 32 + 16, 16)] = m_hi
                u_lo = lax.bitcast_convert_type(m_lo, jnp.int32)
                u_hi = lax.bitcast_convert_type(m_hi, jnp.int32)
                acc16_v[j, pl.ds(g * 16, 16)] = lax.bitwise_or(
                    lax.shift_right_logical(rne_top(u_lo), jnp.int32(16)),
                    lax.bitwise_and(rne_top(u_hi), jnp.int32(_TOPMASK)))
        nb = sid * _W + ci * _C

        @pl.when(cid == 0)
        def _out_a():
            pltpu.sync_copy(accf_v, out_a_hbm.at[pl.ds(nb, _C)])
            pltpu.sync_copy(acc16_v, o16_a_hbm.at[pl.ds(nb, _C)])

        @pl.when(cid == 1)
        def _out_b():
            pltpu.sync_copy(accf_v, out_b_hbm.at[pl.ds(nb, _C)])
            pltpu.sync_copy(acc16_v, o16_b_hbm.at[pl.ds(nb, _C)])

        return carry

    lax.fori_loop(0, _CH, chunk, 0)


def _concat_body(a_ref, b_ref, out_ref):
    out_ref[:, : _D // 2] = a_ref[...]
    out_ref[:, _D // 2:] = b_ref[...]


def _concat(a, b):
    return pl.pallas_call(
        _concat_body,
        out_shape=jax.ShapeDtypeStruct((_NP, _D), jnp.float32),
    )(a, b)


@functools.partial(
    pl.kernel, mesh=_mesh,
    out_type=jax.ShapeDtypeStruct((_B, 16), jnp.float32),
    scratch_types=[
        pltpu.VMEM((_PCH, _PC), jnp.int32),
        pltpu.VMEM((_PCH, _PC), jnp.int32),
        pltpu.VMEM((2, 4, _PC, _D), jnp.float32),
        pltpu.VMEM((_PC, 16), jnp.float32),
        pltpu.SemaphoreType.DMA,
        pltpu.SemaphoreType.DMA,
    ],
)
def _pairs(src_hbm, dst_hbm, h1_hbm, h2_hbm, out_hbm,
           sidx_v, didx_v, rows_v, out_v, sem0, sem1):
    wid = lax.axis_index("s") * 2 + lax.axis_index("c")
    pltpu.sync_copy(src_hbm.at[wid], sidx_v)
    pltpu.sync_copy(dst_hbm.at[wid], didx_v)
    sems = (sem0, sem1)

    def fire(ci, b):
        sem = sems[b]
        pltpu.async_copy(h1_hbm.at[sidx_v.at[ci]], rows_v.at[b, 0], sem)
        pltpu.async_copy(h1_hbm.at[didx_v.at[ci]], rows_v.at[b, 1], sem)
        pltpu.async_copy(h2_hbm.at[sidx_v.at[ci]], rows_v.at[b, 2], sem)
        pltpu.async_copy(h2_hbm.at[didx_v.at[ci]], rows_v.at[b, 3], sem)

    def drain(ci, b):
        sem = sems[b]
        pltpu.make_async_copy(h1_hbm.at[sidx_v.at[ci]], rows_v.at[b, 0], sem).wait()
        pltpu.make_async_copy(h1_hbm.at[didx_v.at[ci]], rows_v.at[b, 1], sem).wait()
        pltpu.make_async_copy(h2_hbm.at[sidx_v.at[ci]], rows_v.at[b, 2], sem).wait()
        pltpu.make_async_copy(h2_hbm.at[didx_v.at[ci]], rows_v.at[b, 3], sem).wait()

    fire(0, 0)
    fire(1, 1)
    for ci in range(_PCH):
        b = ci % 2
        drain(ci, b)

        def pstep(p, carry2):
            acc = jnp.zeros((16,), jnp.float32)
            for g in range(8):
                sl = pl.ds(g * 16, 16)
                v1 = rows_v[b, 0, p, sl] - rows_v[b, 1, p, sl]
                acc = acc + v1 * v1
                v2 = rows_v[b, 2, p, sl] - rows_v[b, 3, p, sl]
                acc = acc + v2 * v2
            out_v[p, :] = acc
            return carry2

        lax.fori_loop(0, _PC, pstep, 0)
        pltpu.sync_copy(out_v, out_hbm.at[pl.ds(wid * _PPW + ci * _PC, _PC)])
        if ci + 2 < _PCH:
            fire(ci + 2, b)


def _bce_body(d16_ref, lbl_ref, out_ref):
    dsum = jnp.sum(d16_ref[...], axis=1, keepdims=True) * (1.0 / (_D * 2))
    p = jnp.exp(-dsum)
    lbl = lbl_ref[...]
    eps = 1e-7
    t = lbl * jnp.log(p + eps) + (1.0 - lbl) * jnp.log(1.0 - p + eps)
    out_ref[...] = (-jnp.mean(t)).reshape(1, 1)


def kernel(pairs, labels, neighbors, embedding_state):
    nbr3 = jnp.pad(neighbors, ((0, _NP - _N), (0, 0))).reshape(_NT, _CH, _RC)
    emb16 = jnp.pad(embedding_state, ((0, _NP - _N), (0, 0))).astype(
        jnp.bfloat16)
    emb_pk = lax.bitcast_convert_type(
        emb16.reshape(_NP, _D // 2, 2), jnp.int32)
    tbl_a = emb_pk[:, :_HW]
    tbl_b = emb_pk[:, _HW:]

    def _step(tbls, _):
        ta, tb = tbls
        h_a, h_b, t16a, t16b = _layer_k(nbr3, ta, tb)
        return (t16a, t16b), (h_a, h_b)

    _, (h_as, h_bs) = lax.scan(_step, (tbl_a, tbl_b), None, length=2)
    h1 = _concat(h_as[0], h_bs[0])
    h2 = _concat(h_as[1], h_bs[1])
    src = pairs[:, 0].reshape(_NW, _PCH, _PC)
    dst = pairs[:, 1].reshape(_NW, _PCH, _PC)
    d16 = _pairs(src, dst, h1, h2)
    lblf = labels.astype(jnp.float32).reshape(_B, 1)
    loss = pl.pallas_call(
        _bce_body,
        out_shape=jax.ShapeDtypeStruct((1, 1), jnp.float32),
    )(d16, lblf)
    return loss.reshape(())


# speedup vs baseline: 3.2155x; 1.0071x over previous
import functools

import jax
import jax.numpy as jnp
from jax import lax
from jax.experimental import pallas as pl
from jax.experimental.pallas import tpu as pltpu
from jax.experimental.pallas import tpu_sc as plsc

_N = 10000
_D = 128
_K = 32
_B = 8192
_NT = 16
_W = 640
_NP = _NT * _W
_HW = _D // 4
_C = 4
_RC = _C * _K
_CH = _W // _C
_NW = 32
_PPW = _B // _NW
_PC = 64
_PCH = _PPW // _PC

_mesh = plsc.VectorSubcoreMesh(core_axis_name="c", subcore_axis_name="s")

_TOPMASK = -65536


@functools.partial(
    pl.kernel, mesh=_mesh,
    out_type=[jax.ShapeDtypeStruct((_NP, _D // 2), jnp.float32),
              jax.ShapeDtypeStruct((_NP, _D // 2), jnp.float32),
              jax.ShapeDtypeStruct((_NP, _HW), jnp.int32),
              jax.ShapeDtypeStruct((_NP, _HW), jnp.int32)],
    scratch_types=[
        pltpu.VMEM((_CH, _RC), jnp.int32),
        pltpu.VMEM((_RC, _HW), jnp.int32),
        pltpu.VMEM((_C, _D // 2), jnp.float32),
        pltpu.VMEM((_C, _HW), jnp.int32),
        pltpu.VMEM_SHARED((_NP, _HW), jnp.int32),
        pltpu.SemaphoreType.DMA,
    ],
)
def _layer_k(nbr_hbm, tbl_a_hbm, tbl_b_hbm,
             out_a_hbm, out_b_hbm, o16_a_hbm, o16_b_hbm,
             idx_v, rows_v, accf_v, acc16_v, tbl_sh, sem):
    cid = lax.axis_index("c")
    sid = lax.axis_index("s")

    @pl.when(cid == 0)
    def _stage_a():
        pltpu.sync_copy(tbl_a_hbm.at[pl.ds(sid * _W, _W)],
                        tbl_sh.at[pl.ds(sid * _W, _W)])

    @pl.when(cid == 1)
    def _stage_b():
        pltpu.sync_copy(tbl_b_hbm.at[pl.ds(sid * _W, _W)],
                        tbl_sh.at[pl.ds(sid * _W, _W)])

    pltpu.sync_copy(nbr_hbm.at[sid], idx_v)
    plsc.subcore_barrier()

    def rne_top(u):
        odd = lax.bitwise_and(
            lax.shift_right_logical(u, jnp.int32(16)), jnp.int32(1))
        return u + jnp.int32(0x7FFF) + odd

    def chunk(ci, carry):
        pltpu.async_copy(tbl_sh.at[idx_v.at[ci]], rows_v, sem).wait()
        for j in range(_C):
            for g in range(2):
                acc_lo = jnp.zeros((16,), jnp.float32)
                acc_hi = jnp.zeros((16,), jnp.float32)
                for k in range(_K):
                    w = rows_v[j * _K + k, pl.ds(g * 16, 16)]
                    acc_lo = acc_lo + lax.bitcast_convert_type(
                        lax.shift_left(w, jnp.int32(16)), jnp.float32)
                    acc_hi = acc_hi + lax.bitcast_convert_type(
                        lax.bitwise_and(w, jnp.int32(_TOPMASK)), jnp.float32)
                m_lo = acc_lo * (1.0 / _K)
                m_hi = acc_hi * (1.0 / _K)
                accf_v[j, pl.ds(g * 32, 16)] = m_lo
                accf_v[j, pl.ds(g * 32 + 16, 16)] = m_hi
                u_lo = lax.bitcast_convert_type(m_lo, jnp.int32)
                u_hi = lax.bitcast_convert_type(m_hi, jnp.int32)
                acc16_v[j, pl.ds(g * 16, 16)] = lax.bitwise_or(
                    lax.shift_right_logical(rne_top(u_lo), jnp.int32(16)),
                    lax.bitwise_and(rne_top(u_hi), jnp.int32(_TOPMASK)))
        nb = sid * _W + ci * _C

        @pl.when(cid == 0)
        def _out_a():
            pltpu.sync_copy(accf_v, out_a_hbm.at[pl.ds(nb, _C)])
            pltpu.sync_copy(acc16_v, o16_a_hbm.at[pl.ds(nb, _C)])

        @pl.when(cid == 1)
        def _out_b():
            pltpu.sync_copy(accf_v, out_b_hbm.at[pl.ds(nb, _C)])
            pltpu.sync_copy(acc16_v, o16_b_hbm.at[pl.ds(nb, _C)])

        return carry

    lax.fori_loop(0, _CH, chunk, 0)


def _concat_body(a1_ref, b1_ref, a2_ref, b2_ref, o1_ref, o2_ref):
    o1_ref[:, : _D // 2] = a1_ref[...]
    o1_ref[:, _D // 2:] = b1_ref[...]
    o2_ref[:, : _D // 2] = a2_ref[...]
    o2_ref[:, _D // 2:] = b2_ref[...]


def _concat2(a1, b1, a2, b2):
    return pl.pallas_call(
        _concat_body,
        out_shape=[jax.ShapeDtypeStruct((_NP, _D), jnp.float32),
                   jax.ShapeDtypeStruct((_NP, _D), jnp.float32)],
    )(a1, b1, a2, b2)


@functools.partial(
    pl.kernel, mesh=_mesh,
    out_type=jax.ShapeDtypeStruct((_B, 16), jnp.float32),
    scratch_types=[
        pltpu.VMEM((_PCH, _PC), jnp.int32),
        pltpu.VMEM((_PCH, _PC), jnp.int32),
        pltpu.VMEM((2, 4, _PC, _D), jnp.float32),
        pltpu.VMEM((_PC, 16), jnp.float32),
        pltpu.SemaphoreType.DMA,
        pltpu.SemaphoreType.DMA,
    ],
)
def _pairs(src_hbm, dst_hbm, h1_hbm, h2_hbm, out_hbm,
           sidx_v, didx_v, rows_v, out_v, sem0, sem1):
    wid = lax.axis_index("s") * 2 + lax.axis_index("c")
    pltpu.sync_copy(src_hbm.at[wid], sidx_v)
    pltpu.sync_copy(dst_hbm.at[wid], didx_v)
    sems = (sem0, sem1)

    def fire(ci, b):
        sem = sems[b]
        pltpu.async_copy(h1_hbm.at[sidx_v.at[ci]], rows_v.at[b, 0], sem)
        pltpu.async_copy(h1_hbm.at[didx_v.at[ci]], rows_v.at[b, 1], sem)
        pltpu.async_copy(h2_hbm.at[sidx_v.at[ci]], rows_v.at[b, 2], sem)
        pltpu.async_copy(h2_hbm.at[didx_v.at[ci]], rows_v.at[b, 3], sem)

    def drain(ci, b):
        sem = sems[b]
        pltpu.make_async_copy(h1_hbm.at[sidx_v.at[ci]], rows_v.at[b, 0], sem).wait()
        pltpu.make_async_copy(h1_hbm.at[didx_v.at[ci]], rows_v.at[b, 1], sem).wait()
        pltpu.make_async_copy(h2_hbm.at[sidx_v.at[ci]], rows_v.at[b, 2], sem).wait()
        pltpu.make_async_copy(h2_hbm.at[didx_v.at[ci]], rows_v.at[b, 3], sem).wait()

    fire(0, 0)
    fire(1, 1)
    for ci in range(_PCH):
        b = ci % 2
        drain(ci, b)

        def pstep(p, carry2):
            acc = jnp.zeros((16,), jnp.float32)
            for g in range(8):
                sl = pl.ds(g * 16, 16)
                v1 = rows_v[b, 0, p, sl] - rows_v[b, 1, p, sl]
                acc = acc + v1 * v1
                v2 = rows_v[b, 2, p, sl] - rows_v[b, 3, p, sl]
                acc = acc + v2 * v2
            out_v[p, :] = acc
            return carry2

        lax.fori_loop(0, _PC, pstep, 0)
        pltpu.sync_copy(out_v, out_hbm.at[pl.ds(wid * _PPW + ci * _PC, _PC)])
        if ci + 2 < _PCH:
            fire(ci + 2, b)


def _bce_body(d16_ref, lbl_ref, out_ref):
    dsum = jnp.sum(d16_ref[...], axis=1, keepdims=True) * (1.0 / (_D * 2))
    p = jnp.exp(-dsum)
    lbl = lbl_ref[...]
    eps = 1e-7
    t = lbl * jnp.log(p + eps) + (1.0 - lbl) * jnp.log(1.0 - p + eps)
    out_ref[...] = (-jnp.mean(t)).reshape(1, 1)


def kernel(pairs, labels, neighbors, embedding_state):
    nbr3 = jnp.pad(neighbors, ((0, _NP - _N), (0, 0))).reshape(_NT, _CH, _RC)
    emb16 = jnp.pad(embedding_state, ((0, _NP - _N), (0, 0))).astype(
        jnp.bfloat16)
    emb_pk = lax.bitcast_convert_type(
        emb16.reshape(_NP, _D // 2, 2), jnp.int32)
    tbl_a = emb_pk[:, :_HW]
    tbl_b = emb_pk[:, _HW:]

    def _step(tbls, _):
        ta, tb = tbls
        h_a, h_b, t16a, t16b = _layer_k(nbr3, ta, tb)
        return (t16a, t16b), (h_a, h_b)

    _, (h_as, h_bs) = lax.scan(_step, (tbl_a, tbl_b), None, length=2)
    h1, h2 = _concat2(h_as[0], h_bs[0], h_as[1], h_bs[1])
    src = pairs[:, 0].reshape(_NW, _PCH, _PC)
    dst = pairs[:, 1].reshape(_NW, _PCH, _PC)
    d16 = _pairs(src, dst, h1, h2)
    lblf = labels.astype(jnp.float32).reshape(_B, 1)
    loss = pl.pallas_call(
        _bce_body,
        out_shape=jax.ShapeDtypeStruct((1, 1), jnp.float32),
    )(d16, lblf)
    return loss.reshape(())
